# baseline (speedup 1.0000x reference)
import jax
import jax.numpy as jnp
import numpy as np
from jax.experimental import pallas as pl
from jax.experimental.pallas import tpu as pltpu

_K = 20
_TJ = 128
_F = 128
_R = 2


def _topk_gather_kernel(fa_ref, fb_ref, sa_ref, sb_ref, tio_ref,
                        sela_ref, selb_ref, sma_ref, smb_ref, msq_ref):
    nr, ti, tj = msq_ref.shape
    xa = fa_ref[...]
    xb = fb_ref[...]
    msq_ref[0:_R] = jnp.sum(xa * xa, axis=3)
    msq_ref[_R:] = jnp.sum(xb * xb, axis=3)
    tio = tio_ref[0]
    bigf = jnp.float32(ti * tj)
    one = jnp.float32(1.0)
    zero = jnp.float32(0.0)

    mag4 = jnp.sqrt(msq_ref[...])
    hits4 = jnp.zeros((nr, ti, tj), dtype=jnp.bool_)
    flats = [None] * _K
    for it in range(_K):
        m4 = jnp.max(mag4, axis=(1, 2), keepdims=True)
        hitv4 = mag4 == m4
        flats[it] = jnp.min(jnp.where(hitv4, tio[None], bigf),
                            axis=(1, 2), keepdims=True)
        hits4 = jnp.logical_or(hits4, hitv4)
        mag4 = jnp.where(hitv4, jnp.float32(-1.0), mag4)
    for r in range(nr):
        feat = fa_ref if r < _R else fb_ref
        sel = sela_ref if r < _R else selb_ref
        rr = r if r < _R else r - _R
        for it in range(_K):
            flat = flats[it][r, 0, 0].astype(jnp.int32)
            i = flat // tj
            j = flat - i * tj
            sel[rr, pl.ds(it, 1), :] = feat[rr, pl.ds(i, 1), pl.ds(j, 1), :][0]
    ssa = jnp.sum(jnp.where(hits4[0:_R], sa_ref[...], zero),
                  axis=(1, 2), keepdims=True)
    ssb = jnp.sum(jnp.where(hits4[_R:], sb_ref[...], zero),
                  axis=(1, 2), keepdims=True)
    sma_ref[...] = ssa / jnp.float32(_K)
    smb_ref[...] = ssb / jnp.float32(_K)

    totals = jnp.sum(jnp.where(hits4, one, zero), axis=(1, 2), keepdims=True)
    for r in range(nr):
        total = totals[r, 0, 0]
        feat = fa_ref if r < _R else fb_ref
        sel = sela_ref if r < _R else selb_ref
        sc = sa_ref if r < _R else sb_ref
        sm = sma_ref if r < _R else smb_ref
        rr = r if r < _R else r - _R

        def _exact(r=r, rr=rr, feat=feat, sel=sel, sc=sc, sm=sm):
            mag = jnp.sqrt(msq_ref[r])
            hacc = jnp.zeros((ti, tj), dtype=jnp.bool_)
            for it in range(_K):
                m = jnp.max(mag, axis=(0, 1), keepdims=True)
                fv = jnp.min(jnp.where(mag == m, tio, bigf),
                             axis=(0, 1), keepdims=True)
                hit = tio == fv
                hacc = jnp.logical_or(hacc, hit)
                mag = jnp.where(hit, jnp.float32(-1.0), mag)
                flat = fv[0, 0].astype(jnp.int32)
                i = flat // tj
                j = flat - i * tj
                sel[rr, pl.ds(it, 1), :] = (
                    feat[rr, pl.ds(i, 1), pl.ds(j, 1), :][0])
            ssum = jnp.sum(jnp.where(hacc, sc[rr], zero),
                           axis=(0, 1), keepdims=True)
            sm[rr] = ssum / jnp.float32(_K)

        pl.when(total != jnp.float32(_K))(_exact)


@jax.jit
def _run(xa, xb, sca, scb, tio):
    hb, ti, tj, f = xa.shape
    return pl.pallas_call(
        _topk_gather_kernel,
        grid=(hb // _R,),
        in_specs=[
            pl.BlockSpec((_R, ti, tj, f), lambda b: (b, 0, 0, 0)),
            pl.BlockSpec((_R, ti, tj, f), lambda b: (b, 0, 0, 0)),
            pl.BlockSpec((_R, ti, tj), lambda b: (b, 0, 0)),
            pl.BlockSpec((_R, ti, tj), lambda b: (b, 0, 0)),
            pl.BlockSpec((1, ti, tj), lambda b: (0, 0, 0)),
        ],
        out_specs=[
            pl.BlockSpec((_R, _K, f), lambda b: (b, 0, 0)),
            pl.BlockSpec((_R, _K, f), lambda b: (b, 0, 0)),
            pl.BlockSpec((_R, 1, 1), lambda b: (b, 0, 0)),
            pl.BlockSpec((_R, 1, 1), lambda b: (b, 0, 0)),
        ],
        out_shape=[
            jax.ShapeDtypeStruct((hb, _K, f), jnp.float32),
            jax.ShapeDtypeStruct((hb, _K, f), jnp.float32),
            jax.ShapeDtypeStruct((hb, 1, 1), jnp.float32),
            jax.ShapeDtypeStruct((hb, 1, 1), jnp.float32),
        ],
        scratch_shapes=[pltpu.VMEM((2 * _R, ti, tj), jnp.float32)],
        compiler_params=pltpu.CompilerParams(
            vmem_limit_bytes=100 * 1024 * 1024),
    )(xa, xb, sca, scb, tio)


def kernel(features, scores, k):
    bc, t, f = features.shape
    half = bc // 2
    ti = t // _TJ
    xr = features.reshape(bc, ti, _TJ, f)
    sc3 = scores.reshape(bc, ti, _TJ)
    tio = jnp.asarray(
        np.arange(t, dtype=np.float32).reshape(1, ti, _TJ))
    sel_n, sel_a, sm_n, sm_a = _run(
        xr[:half], xr[half:], sc3[:half], sc3[half:], tio)
    return (sm_a.reshape(half, 1), sm_n.reshape(half, 1), sel_a, sel_n)


# speedup vs baseline: 2.3283x; 2.3283x over previous
import jax
import jax.numpy as jnp
import numpy as np
from jax.experimental import pallas as pl
from jax.experimental.pallas import tpu as pltpu

_K = 20
_TJ = 128
_F = 128
_R = 4


def _topk_gather_kernel(feat_ref, sc_ref, tio_ref, sel_ref, smean_ref, msq_ref):
    nr, ti, tj = msq_ref.shape
    x = feat_ref[...]
    msq_ref[...] = jnp.sum(x * x, axis=3)
    tio = tio_ref[0]
    bigf = jnp.float32(ti * tj)
    one = jnp.float32(1.0)
    zero = jnp.float32(0.0)

    mag4 = jnp.sqrt(msq_ref[...])
    hits4 = jnp.zeros((nr, ti, tj), dtype=jnp.bool_)
    flats = [None] * _K
    for it in range(_K):
        m4 = jnp.max(mag4, axis=(1, 2), keepdims=True)
        hitv4 = mag4 == m4
        flats[it] = jnp.min(jnp.where(hitv4, tio[None], bigf),
                            axis=(1, 2), keepdims=True)
        hits4 = jnp.logical_or(hits4, hitv4)
        mag4 = jnp.where(hitv4, jnp.float32(-1.0), mag4)
    for r in range(nr):
        for it in range(_K):
            flat = flats[it][r, 0, 0].astype(jnp.int32)
            i = flat // tj
            j = flat - i * tj
            sel_ref[r, pl.ds(it, 1), :] = feat_ref[r, pl.ds(i, 1), pl.ds(j, 1), :][0]
    ssum4 = jnp.sum(jnp.where(hits4, sc_ref[...], zero),
                    axis=(1, 2), keepdims=True)
    smean_ref[...] = ssum4 / jnp.float32(_K)

    totals = jnp.sum(jnp.where(hits4, one, zero), axis=(1, 2), keepdims=True)
    for r in range(nr):
        total = totals[r, 0, 0]

        def _exact(r=r):
            mag = jnp.sqrt(msq_ref[r])
            hacc = jnp.zeros((ti, tj), dtype=jnp.bool_)
            for it in range(_K):
                m = jnp.max(mag, axis=(0, 1), keepdims=True)
                fv = jnp.min(jnp.where(mag == m, tio, bigf),
                             axis=(0, 1), keepdims=True)
                hit = tio == fv
                hacc = jnp.logical_or(hacc, hit)
                mag = jnp.where(hit, jnp.float32(-1.0), mag)
                flat = fv[0, 0].astype(jnp.int32)
                i = flat // tj
                j = flat - i * tj
                sel_ref[r, pl.ds(it, 1), :] = (
                    feat_ref[r, pl.ds(i, 1), pl.ds(j, 1), :][0])
            ssum = jnp.sum(jnp.where(hacc, sc_ref[r], zero),
                           axis=(0, 1), keepdims=True)
            smean_ref[r] = ssum / jnp.float32(_K)

        pl.when(total != jnp.float32(_K))(_exact)


@jax.jit
def _run(xr, sc3, tio):
    bc, ti, tj, f = xr.shape
    return pl.pallas_call(
        _topk_gather_kernel,
        grid=(bc // _R,),
        in_specs=[
            pl.BlockSpec((_R, ti, tj, f), lambda b: (b, 0, 0, 0)),
            pl.BlockSpec((_R, ti, tj), lambda b: (b, 0, 0)),
            pl.BlockSpec((1, ti, tj), lambda b: (0, 0, 0)),
        ],
        out_specs=[
            pl.BlockSpec((_R, _K, f), lambda b: (b, 0, 0)),
            pl.BlockSpec((_R, 1, 1), lambda b: (b, 0, 0)),
        ],
        out_shape=[
            jax.ShapeDtypeStruct((bc, _K, f), jnp.float32),
            jax.ShapeDtypeStruct((bc, 1, 1), jnp.float32),
        ],
        scratch_shapes=[pltpu.VMEM((_R, ti, tj), jnp.float32)],
    )(xr, sc3, tio)


def kernel(features, scores, k):
    bc, t, f = features.shape
    half = bc // 2
    ti = t // _TJ
    xr = features.reshape(bc, ti, _TJ, f)
    sc3 = scores.reshape(bc, ti, _TJ)
    tio = jnp.asarray(
        np.arange(t, dtype=np.float32).reshape(1, ti, _TJ))
    sel, smean = _run(xr, sc3, tio)
    smean = smean.reshape(bc, 1)
    return (smean[half:], smean[:half], sel[half:], sel[:half])
